# Initial kernel scaffold; baseline (speedup 1.0000x reference)
import jax
import jax.numpy as jnp
from jax.experimental import pallas as pl


def kernel(pc1, pc2, feat1, feat2, W1_0, b1_0, W1_1, b1_1, W1_2, b1_2, W2_0, b2_0, W2_1, b2_1):
    raise NotImplementedError("write your pallas kernel here")


# speedup vs baseline: 16.0832x; 16.0832x over previous
import functools

import jax
import jax.numpy as jnp
from jax.experimental import pallas as pl
from jax.experimental.pallas import tpu as pltpu
from jax.experimental.pallas import tpu_sc as plsc

_N = 4096
_K = 16
_BQ = 256
_BP = 1024
_BR = 256
_GW = 128

def _leaky(x):
    return jnp.where(x > 0, x, 0.1 * x)


def _b16(x):
    return x.astype(jnp.bfloat16).astype(jnp.float32)


def _dot16(a, w):
    return jnp.dot(a.astype(jnp.bfloat16), w.astype(jnp.bfloat16),
                   preferred_element_type=jnp.float32)


def _topk_body(xq_ref, xk_ref, idx_ref):
    b = pl.program_id(0)
    xq = xq_ref[0]
    iota = jax.lax.broadcasted_iota(jnp.int32, (_BQ, _N), 1)
    s1 = xq[:, 0:1] ** 2 + xq[:, 1:2] ** 2 + xq[:, 2:3] ** 2
    cr = None
    s2 = None
    for c in range(3):
        xkc = xk_ref[0, c:c + 1, :]
        xqc = xq[:, c:c + 1]
        t = _b16(xqc) * _b16(xkc)
        cr = t if cr is None else cr + t
        s2c = xkc * xkc
        s2 = s2c if s2 is None else s2 + s2c
    d = -2.0 * cr + s1 + s2
    off = b * _N
    for k in range(_K):
        m = jnp.min(d, axis=1, keepdims=True)
        cand = jnp.where(d == m, iota, _N)
        a = jnp.min(cand, axis=1, keepdims=True)
        idx_ref[0, :, k:k + 1] = a + off
        d = jnp.where(iota == a, jnp.float32(jnp.inf), d)


def _topk(xqt, xk):
    B = xqt.shape[0]
    return pl.pallas_call(
        _topk_body,
        grid=(B, _N // _BQ),
        in_specs=[pl.BlockSpec((1, _BQ, 3), lambda b, i: (b, i, 0)),
                  pl.BlockSpec((1, 3, _N), lambda b, i: (b, 0, 0))],
        out_specs=pl.BlockSpec((1, _BQ, _K), lambda b, i: (b, i, 0)),
        out_shape=jax.ShapeDtypeStruct((B, _N, _K), jnp.int32),
    )(xqt, xk)


def _prep_body(p1_ref, p2_ref, x1_ref, x2_ref,
               wa_ref, wb_ref, wc_ref, bias_ref, q_ref, kv_ref):
    wc = _b16(wc_ref[...])

    def xterm(x):
        xb = _b16(x)
        return (xb[:, 0:1] * wc[0:1, :] + xb[:, 1:2] * wc[1:2, :]
                + xb[:, 2:3] * wc[2:3, :])

    q = _dot16(p1_ref[0], wa_ref[...])
    q = q - xterm(x1_ref[0]) + bias_ref[...]
    kv = _dot16(p2_ref[0], wb_ref[...])
    kv = kv + xterm(x2_ref[0])
    q_ref[0] = q
    kv_ref[0] = kv


def _prep(p1t, p2t, x1t, x2t, wa, wb, wc, bias):
    B, _, D = p1t.shape
    C = wa.shape[1]
    return pl.pallas_call(
        _prep_body,
        grid=(B, _N // _BP),
        in_specs=[pl.BlockSpec((1, _BP, D), lambda b, i: (b, i, 0)),
                  pl.BlockSpec((1, _BP, D), lambda b, i: (b, i, 0)),
                  pl.BlockSpec((1, _BP, 3), lambda b, i: (b, i, 0)),
                  pl.BlockSpec((1, _BP, 3), lambda b, i: (b, i, 0)),
                  pl.BlockSpec((D, C), lambda b, i: (0, 0)),
                  pl.BlockSpec((D, C), lambda b, i: (0, 0)),
                  pl.BlockSpec((3, C), lambda b, i: (0, 0)),
                  pl.BlockSpec((1, C), lambda b, i: (0, 0))],
        out_specs=[pl.BlockSpec((1, _BP, C), lambda b, i: (b, i, 0)),
                   pl.BlockSpec((1, _BP, C), lambda b, i: (b, i, 0))],
        out_shape=[jax.ShapeDtypeStruct((B, _N, C), jnp.float32),
                   jax.ShapeDtypeStruct((B, _N, C), jnp.float32)],
    )(p1t, p2t, x1t, x2t, wa, wb, wc, bias)


def _sc_gather(table, idx_flat):
    NI = idx_flat.shape[1]
    C = table.shape[1]
    mesh = plsc.VectorSubcoreMesh(core_axis_name="core",
                                  subcore_axis_name="subcore")

    @functools.partial(
        pl.kernel,
        out_type=jax.ShapeDtypeStruct((NI, C), table.dtype),
        mesh=mesh)
    def kern(x_hbm, i_hbm, o_hbm):
        def body(i_vmem, o_vmem):
            pltpu.sync_copy(x_hbm.at[i_vmem.at[0]], o_vmem)

        pltpu.emit_pipeline(
            body,
            grid=(NI // _GW,),
            in_specs=[pl.BlockSpec((1, _GW), index_map=lambda i: (0, i))],
            out_specs=[pl.BlockSpec((_GW, C), index_map=lambda i: (i, 0))],
            core_axis_name=("core", "subcore"),
            dimension_semantics=(pltpu.PARALLEL,),
        )(i_hbm, o_hbm)

    return kern(table, idx_flat)


def _mlp_body(nl, g_ref, q_ref, *rest):
    wrefs = rest[:2 * nl]
    out_nc_ref, out_cn_ref = rest[2 * nl], rest[2 * nl + 1]
    g = g_ref[0]
    q = q_ref[0]
    C = q.shape[1]
    a = g.reshape(_BR, _K, C) + q[:, None, :]
    a = _leaky(a).reshape(_BR * _K, C)
    for i in range(nl):
        w = wrefs[2 * i][...]
        bb = wrefs[2 * i + 1][...]
        a = _dot16(a, w) + bb
        a = _leaky(a)
    r = jnp.max(a.reshape(_BR, _K, C), axis=1)
    out_nc_ref[0] = r
    out_cn_ref[0] = r.T


def _mlp(g, q, layers):
    B = q.shape[0]
    C = q.shape[2]
    nl = len(layers)
    wspecs = []
    wargs = []
    for (w, bb) in layers:
        wspecs.append(pl.BlockSpec((C, C), lambda b, i: (0, 0)))
        wspecs.append(pl.BlockSpec((1, C), lambda b, i: (0, 0)))
        wargs.extend([w, bb])
    return pl.pallas_call(
        functools.partial(_mlp_body, nl),
        grid=(B, _N // _BR),
        in_specs=[pl.BlockSpec((1, _BR * _K, C), lambda b, i: (b, i, 0)),
                  pl.BlockSpec((1, _BR, C), lambda b, i: (b, i, 0))]
                 + wspecs,
        out_specs=[pl.BlockSpec((1, _BR, C), lambda b, i: (b, i, 0)),
                   pl.BlockSpec((1, C, _BR), lambda b, i: (b, 0, i))],
        out_shape=[jax.ShapeDtypeStruct((B, _N, C), jnp.float32),
                   jax.ShapeDtypeStruct((B, C, _N), jnp.float32)],
    )(g, q, *wargs)


def _stage(p1t, p2t, x1t, x2t, idx, w0, b0, tail):
    B, _, D = p1t.shape
    wa = w0[:, :D].T
    wb = w0[:, D:2 * D].T
    wc = w0[:, 2 * D:].T
    q, kv = _prep(p1t, p2t, x1t, x2t, wa, wb, wc, b0.reshape(1, -1))
    C = wa.shape[1]
    g = _sc_gather(kv.reshape(B * _N, C), idx.reshape(1, B * _N * _K))
    layers = [(w.T, bb.reshape(1, -1)) for (w, bb) in tail]
    return _mlp(g.reshape(B, _N * _K, C), q, layers)


def kernel(pc1, pc2, feat1, feat2, W1_0, b1_0, W1_1, b1_1, W1_2, b1_2,
           W2_0, b2_0, W2_1, b2_1):
    x1t = jnp.transpose(pc1, (0, 2, 1))
    x2t = jnp.transpose(pc2, (0, 2, 1))
    p1t = jnp.transpose(feat1, (0, 2, 1))
    p2t = jnp.transpose(feat2, (0, 2, 1))

    idx12 = _topk(x1t, pc2)
    idx21 = _topk(x2t, pc1)

    tail1 = [(W1_1, b1_1), (W1_2, b1_2)]
    f1n_nc, f1n_cn = _stage(p1t, p2t, x1t, x2t, idx12, W1_0, b1_0, tail1)
    f2n_nc, f2n_cn = _stage(p2t, p1t, x2t, x1t, idx21, W1_0, b1_0, tail1)
    _, f1f_cn = _stage(f1n_nc, f2n_nc, x1t, x2t, idx12, W2_0, b2_0,
                       [(W2_1, b2_1)])
    return (f1n_cn, f2n_cn, f1f_cn)
